# baseline (speedup 1.0000x reference)
import functools

import jax
import jax.numpy as jnp
from jax import lax
from jax.experimental import pallas as pl
from jax.experimental.pallas import tpu as pltpu

R = 128
V = 100000
BP = 2048
NBLK = (V + BP - 1) // BP


def _tc_body(lt_ref, vals_ref, idxs_ref, m_ref, mi_ref):
    i = pl.program_id(0)

    @pl.when(i == 0)
    def _():
        m_ref[...] = jnp.full((8, R), -jnp.inf, jnp.float32)
        mi_ref[...] = jnp.zeros((8, R), jnp.int32)

    base = i * BP
    pos8 = lax.broadcasted_iota(jnp.int32, (8, R), 0)
    m = m_ref[...]
    mi = mi_ref[...]
    for s in range(BP // 8):
        v = lt_ref[pl.ds(s * 8, 8), :]
        pos = pos8 + (base + s * 8)
        cmp = (v > m) & (pos < V)
        m = jnp.where(cmp, v, m)
        mi = jnp.where(cmp, pos, mi)
    m_ref[...] = m
    mi_ref[...] = mi

    @pl.when(i == NBLK - 1)
    def _():
        vals_ref[...] = m
        idxs_ref[...] = mi


@functools.partial(jax.jit)
def _tc_argmax(lt):
    return pl.pallas_call(
        _tc_body,
        grid=(NBLK,),
        in_specs=[pl.BlockSpec((BP, R), lambda i: (i, 0))],
        out_specs=(
            pl.BlockSpec((8, R), lambda i: (0, 0)),
            pl.BlockSpec((8, R), lambda i: (0, 0)),
        ),
        out_shape=(
            jax.ShapeDtypeStruct((8, R), jnp.float32),
            jax.ShapeDtypeStruct((8, R), jnp.int32),
        ),
        scratch_shapes=[
            pltpu.VMEM((8, R), jnp.float32),
            pltpu.VMEM((8, R), jnp.int32),
        ],
        compiler_params=pltpu.CompilerParams(
            dimension_semantics=("arbitrary",)),
    )(lt)


def kernel(logits):
    vals, idxs = _tc_argmax(logits.T)
    m = jnp.max(vals, axis=0)
    cand = jnp.where(vals == m[None, :], idxs, jnp.int32(V))
    return jnp.min(cand, axis=0)


# speedup vs baseline: 1.0747x; 1.0747x over previous
import functools

import jax
import jax.numpy as jnp
from jax import lax
from jax.experimental import pallas as pl
from jax.experimental.pallas import tpu as pltpu

R = 128
V = 100000
BP = 2048
NBLK = (V + BP - 1) // BP


U = 8


def _tc_body(lt_ref, vals_ref, idxs_ref, m_ref, mi_ref):
    i = pl.program_id(0)

    @pl.when(i == 0)
    def _():
        m_ref[...] = jnp.full((8 * U, R), -jnp.inf, jnp.float32)
        mi_ref[...] = jnp.zeros((8 * U, R), jnp.int32)

    base = i * BP
    pos8 = lax.broadcasted_iota(jnp.int32, (8, R), 0)
    ms = [m_ref[pl.ds(8 * k, 8), :] for k in range(U)]
    mis = [mi_ref[pl.ds(8 * k, 8), :] for k in range(U)]
    for s in range(BP // 8):
        k = s % U
        v = lt_ref[pl.ds(s * 8, 8), :]
        pos = pos8 + (base + s * 8)
        cmp = (v > ms[k]) & (pos < V)
        ms[k] = jnp.where(cmp, v, ms[k])
        mis[k] = jnp.where(cmp, pos, mis[k])
    for k in range(U):
        m_ref[pl.ds(8 * k, 8), :] = ms[k]
        mi_ref[pl.ds(8 * k, 8), :] = mis[k]

    @pl.when(i == NBLK - 1)
    def _():
        vals_ref[...] = m_ref[...]
        idxs_ref[...] = mi_ref[...]


@functools.partial(jax.jit)
def _tc_argmax(lt):
    return pl.pallas_call(
        _tc_body,
        grid=(NBLK,),
        in_specs=[pl.BlockSpec((BP, R), lambda i: (i, 0))],
        out_specs=(
            pl.BlockSpec((8 * U, R), lambda i: (0, 0)),
            pl.BlockSpec((8 * U, R), lambda i: (0, 0)),
        ),
        out_shape=(
            jax.ShapeDtypeStruct((8 * U, R), jnp.float32),
            jax.ShapeDtypeStruct((8 * U, R), jnp.int32),
        ),
        scratch_shapes=[
            pltpu.VMEM((8 * U, R), jnp.float32),
            pltpu.VMEM((8 * U, R), jnp.int32),
        ],
        compiler_params=pltpu.CompilerParams(
            dimension_semantics=("arbitrary",)),
    )(lt)


def kernel(logits):
    vals, idxs = _tc_argmax(logits.T)
    m = jnp.max(vals, axis=0)
    cand = jnp.where(vals == m[None, :], idxs, jnp.int32(V))
    return jnp.min(cand, axis=0)


# speedup vs baseline: 1.7690x; 1.6461x over previous
import functools

import jax
import jax.numpy as jnp
from jax import lax
from jax.experimental import pallas as pl
from jax.experimental.pallas import tpu as pltpu

R = 128
V = 100000
BP = 8192
NBLK = (V + BP - 1) // BP


U = 8


def _tc_body(lt_ref, vals_ref, idxs_ref, m_ref, mi_ref):
    i = pl.program_id(0)

    @pl.when(i == 0)
    def _():
        m_ref[...] = jnp.full((8 * U, R), -jnp.inf, jnp.float32)
        mi_ref[...] = jnp.zeros((8 * U, R), jnp.int32)

    base = i * BP
    pos8 = lax.broadcasted_iota(jnp.int32, (8, R), 0)
    ms = [m_ref[pl.ds(8 * k, 8), :] for k in range(U)]
    mis = [mi_ref[pl.ds(8 * k, 8), :] for k in range(U)]
    for s in range(BP // 8):
        k = s % U
        v = lt_ref[pl.ds(s * 8, 8), :]
        pos = pos8 + (base + s * 8)
        cmp = (v > ms[k]) & (pos < V)
        ms[k] = jnp.where(cmp, v, ms[k])
        mis[k] = jnp.where(cmp, pos, mis[k])
    for k in range(U):
        m_ref[pl.ds(8 * k, 8), :] = ms[k]
        mi_ref[pl.ds(8 * k, 8), :] = mis[k]

    @pl.when(i == NBLK - 1)
    def _():
        vals_ref[...] = m_ref[...]
        idxs_ref[...] = mi_ref[...]


@functools.partial(jax.jit)
def _tc_argmax(lt):
    return pl.pallas_call(
        _tc_body,
        grid=(NBLK,),
        in_specs=[pl.BlockSpec((BP, R), lambda i: (i, 0))],
        out_specs=(
            pl.BlockSpec((8 * U, R), lambda i: (0, 0)),
            pl.BlockSpec((8 * U, R), lambda i: (0, 0)),
        ),
        out_shape=(
            jax.ShapeDtypeStruct((8 * U, R), jnp.float32),
            jax.ShapeDtypeStruct((8 * U, R), jnp.int32),
        ),
        scratch_shapes=[
            pltpu.VMEM((8 * U, R), jnp.float32),
            pltpu.VMEM((8 * U, R), jnp.int32),
        ],
        compiler_params=pltpu.CompilerParams(
            dimension_semantics=("arbitrary",)),
    )(lt)


def kernel(logits):
    vals, idxs = _tc_argmax(logits.T)
    m = jnp.max(vals, axis=0)
    cand = jnp.where(vals == m[None, :], idxs, jnp.int32(V))
    return jnp.min(cand, axis=0)


# speedup vs baseline: 1.8039x; 1.0197x over previous
import functools

import jax
import jax.numpy as jnp
from jax import lax
from jax.experimental import pallas as pl
from jax.experimental.pallas import tpu as pltpu

R = 128
V = 100000
BP = 16384
NBLK = (V + BP - 1) // BP


U = 8


def _tc_body(lt_ref, vals_ref, idxs_ref, m_ref, mi_ref):
    i = pl.program_id(0)

    @pl.when(i == 0)
    def _():
        m_ref[...] = jnp.full((8 * U, R), -jnp.inf, jnp.float32)
        mi_ref[...] = jnp.zeros((8 * U, R), jnp.int32)

    base = i * BP
    pos8 = lax.broadcasted_iota(jnp.int32, (8, R), 0)
    ms = [m_ref[pl.ds(8 * k, 8), :] for k in range(U)]
    mis = [mi_ref[pl.ds(8 * k, 8), :] for k in range(U)]
    for s in range(BP // 8):
        k = s % U
        v = lt_ref[pl.ds(s * 8, 8), :]
        pos = pos8 + (base + s * 8)
        cmp = (v > ms[k]) & (pos < V)
        ms[k] = jnp.where(cmp, v, ms[k])
        mis[k] = jnp.where(cmp, pos, mis[k])
    for k in range(U):
        m_ref[pl.ds(8 * k, 8), :] = ms[k]
        mi_ref[pl.ds(8 * k, 8), :] = mis[k]

    @pl.when(i == NBLK - 1)
    def _():
        vals_ref[...] = m_ref[...]
        idxs_ref[...] = mi_ref[...]


@functools.partial(jax.jit)
def _tc_argmax(lt):
    return pl.pallas_call(
        _tc_body,
        grid=(NBLK,),
        in_specs=[pl.BlockSpec((BP, R), lambda i: (i, 0))],
        out_specs=(
            pl.BlockSpec((8 * U, R), lambda i: (0, 0)),
            pl.BlockSpec((8 * U, R), lambda i: (0, 0)),
        ),
        out_shape=(
            jax.ShapeDtypeStruct((8 * U, R), jnp.float32),
            jax.ShapeDtypeStruct((8 * U, R), jnp.int32),
        ),
        scratch_shapes=[
            pltpu.VMEM((8 * U, R), jnp.float32),
            pltpu.VMEM((8 * U, R), jnp.int32),
        ],
        compiler_params=pltpu.CompilerParams(
            dimension_semantics=("arbitrary",)),
    )(lt)


def kernel(logits):
    vals, idxs = _tc_argmax(logits.T)
    m = jnp.max(vals, axis=0)
    cand = jnp.where(vals == m[None, :], idxs, jnp.int32(V))
    return jnp.min(cand, axis=0)
